# baseline (speedup 1.0000x reference)
import functools

import jax
import jax.numpy as jnp
from jax import lax
from jax.experimental import pallas as pl
from jax.experimental.pallas import tpu as pltpu
from jax.experimental.pallas import tpu_sc as plsc

N = 10000
NP = 10240
NBLK = 10
BLK = N // NBLK
NW = 32
TPC = 16
ROWS_PER_TILE = NP // TPC
CHW = 128


def _pad_edges(src, dst, e):
    epw = -(-e // NW)
    n_chunk = -(-epw // CHW)
    if n_chunk % 2 == 0:
        n_chunk += 1
    ep = NW * n_chunk * CHW
    pad = ep - e
    src3 = jnp.concatenate([src, jnp.zeros((pad,), jnp.int32)])
    dst3 = jnp.concatenate([dst, jnp.full((pad,), N, dtype=jnp.int32)])
    return src3.reshape(NW, n_chunk, CHW), dst3.reshape(NW, n_chunk, CHW), n_chunk


_SC_PARAMS = pltpu.CompilerParams(use_tc_tiling_on_sc=False)


def _make_deg_kernel(n_chunk):
    mesh = plsc.VectorSubcoreMesh(core_axis_name="c", subcore_axis_name="s")

    @functools.partial(
        pl.kernel,
        out_type=jax.ShapeDtypeStruct((2, NP, 16), jnp.float32),
        mesh=mesh,
        compiler_params=_SC_PARAMS,
        scratch_types=[
            pltpu.VMEM_SHARED((NP, 16), jnp.float32),
            pltpu.VMEM((n_chunk, CHW), jnp.int32),
            pltpu.VMEM((CHW, 16), jnp.float32),
        ],
    )
    def deg_kernel(dst3_hbm, zeros_hbm, out_hbm, acc, dstv, ones_v):
        c = lax.axis_index("c")
        s = lax.axis_index("s")
        wid = c * TPC + s
        pltpu.sync_copy(zeros_hbm, acc.at[pl.ds(s * ROWS_PER_TILE, ROWS_PER_TILE)])
        for r in range(CHW):
            ones_v[r, :] = jnp.ones((16,), jnp.float32)
        plsc.subcore_barrier()
        pltpu.sync_copy(dst3_hbm.at[wid], dstv)

        def chunk(j, carry):
            pltpu.sync_copy(ones_v, acc.at[dstv.at[j]], add=True)
            return carry

        lax.fori_loop(0, n_chunk, chunk, 0)
        plsc.subcore_barrier()
        sl = pl.ds(s * ROWS_PER_TILE, ROWS_PER_TILE)
        pltpu.sync_copy(acc.at[sl], out_hbm.at[c, sl])

    return deg_kernel


def _make_scat_kernel(n_chunk, ch, d):
    mesh = plsc.VectorSubcoreMesh(core_axis_name="c", subcore_axis_name="s")

    @functools.partial(
        pl.kernel,
        out_type=jax.ShapeDtypeStruct((2, NP, d), jnp.float32),
        mesh=mesh,
        compiler_params=_SC_PARAMS,
        scratch_types=[
            pltpu.VMEM_SHARED((NP, d), jnp.float32),
            pltpu.VMEM((n_chunk, ch), jnp.int32),
            pltpu.VMEM((n_chunk, ch), jnp.int32),
            pltpu.VMEM((ch, d), jnp.float32),
            pltpu.VMEM((ch, d), jnp.float32),
            pltpu.SemaphoreType.DMA,
            pltpu.SemaphoreType.DMA,
        ],
    )
    def scat_kernel(hp_hbm, src3_hbm, dst3_hbm, zeros_hbm, out_hbm,
                    acc, srcv, dstv, rows_a, rows_b, sem_a, sem_b):
        c = lax.axis_index("c")
        s = lax.axis_index("s")
        wid = c * TPC + s
        pltpu.sync_copy(zeros_hbm, acc.at[pl.ds(s * ROWS_PER_TILE, ROWS_PER_TILE)])
        plsc.subcore_barrier()
        pltpu.sync_copy(src3_hbm.at[wid], srcv)
        pltpu.sync_copy(dst3_hbm.at[wid], dstv)

        def g_start(j, buf, sem):
            pltpu.async_copy(hp_hbm.at[srcv.at[j]], buf, sem)

        def g_wait(j, buf, sem):
            pltpu.make_async_copy(hp_hbm.at[srcv.at[j]], buf, sem).wait()

        def scat(j, buf):
            pltpu.sync_copy(buf, acc.at[dstv.at[j]], add=True)

        g_start(0, rows_a, sem_a)

        def pair(k, carry):
            ja = 2 * k
            g_wait(ja, rows_a, sem_a)
            g_start(ja + 1, rows_b, sem_b)
            scat(ja, rows_a)
            g_wait(ja + 1, rows_b, sem_b)
            g_start(ja + 2, rows_a, sem_a)
            scat(ja + 1, rows_b)
            return carry

        if n_chunk % 2:
            lax.fori_loop(0, (n_chunk - 1) // 2, pair, 0)
            g_wait(n_chunk - 1, rows_a, sem_a)
            scat(n_chunk - 1, rows_a)
        else:
            lax.fori_loop(0, n_chunk // 2 - 1, pair, 0)
            g_wait(n_chunk - 2, rows_a, sem_a)
            g_start(n_chunk - 1, rows_b, sem_b)
            scat(n_chunk - 2, rows_a)
            g_wait(n_chunk - 1, rows_b, sem_b)
            scat(n_chunk - 1, rows_b)
        plsc.subcore_barrier()
        sl = pl.ds(s * ROWS_PER_TILE, ROWS_PER_TILE)
        pltpu.sync_copy(acc.at[sl], out_hbm.at[c, sl])

    return scat_kernel


def _mm1_body(x_ref, w_ref, b_ref, dinv_ref, o_ref):
    acc = jnp.dot(x_ref[...], w_ref[...], preferred_element_type=jnp.float32)
    o_ref[...] = dinv_ref[...] * (acc + b_ref[...])


def _mm1(x, w1, b1r, dinv_col):
    return pl.pallas_call(
        _mm1_body,
        grid=(NBLK,),
        in_specs=[
            pl.BlockSpec((BLK, 128), lambda i: (i, 0)),
            pl.BlockSpec((128, 128), lambda i: (0, 0)),
            pl.BlockSpec((1, 128), lambda i: (0, 0)),
            pl.BlockSpec((BLK, 1), lambda i: (i, 0)),
        ],
        out_specs=pl.BlockSpec((BLK, 128), lambda i: (i, 0)),
        out_shape=jax.ShapeDtypeStruct((N, 128), jnp.float32),
    )(x, w1, b1r, dinv_col)


def _mm2_body(sa_ref, sb_ref, hp_ref, dinv_ref, w_ref, b_ref, o1_ref, h2_ref):
    smooth = dinv_ref[...] * (sa_ref[...] + sb_ref[...] + hp_ref[...])
    o1 = jnp.maximum(smooth, 0.0)
    o1_ref[...] = o1
    acc = jnp.dot(o1, w_ref[...], preferred_element_type=jnp.float32)
    h2_ref[...] = dinv_ref[...] * (acc + b_ref[...])


def _mm2(s1a, s1b, hp1, dinv_col, w2, b2r):
    return pl.pallas_call(
        _mm2_body,
        grid=(NBLK,),
        in_specs=[
            pl.BlockSpec((BLK, 128), lambda i: (i, 0)),
            pl.BlockSpec((BLK, 128), lambda i: (i, 0)),
            pl.BlockSpec((BLK, 128), lambda i: (i, 0)),
            pl.BlockSpec((BLK, 1), lambda i: (i, 0)),
            pl.BlockSpec((128, 64), lambda i: (0, 0)),
            pl.BlockSpec((1, 64), lambda i: (0, 0)),
        ],
        out_specs=[
            pl.BlockSpec((BLK, 128), lambda i: (i, 0)),
            pl.BlockSpec((BLK, 64), lambda i: (i, 0)),
        ],
        out_shape=[
            jax.ShapeDtypeStruct((N, 128), jnp.float32),
            jax.ShapeDtypeStruct((N, 64), jnp.float32),
        ],
    )(s1a, s1b, hp1, dinv_col, w2, b2r)


def _fin_body(sa_ref, sb_ref, hp_ref, dinv_ref, o_ref):
    o_ref[...] = dinv_ref[...] * (sa_ref[...] + sb_ref[...] + hp_ref[...])


def _fin(s2a, s2b, hp2, dinv_col):
    return pl.pallas_call(
        _fin_body,
        grid=(NBLK,),
        in_specs=[
            pl.BlockSpec((BLK, 64), lambda i: (i, 0)),
            pl.BlockSpec((BLK, 64), lambda i: (i, 0)),
            pl.BlockSpec((BLK, 64), lambda i: (i, 0)),
            pl.BlockSpec((BLK, 1), lambda i: (i, 0)),
        ],
        out_specs=pl.BlockSpec((BLK, 64), lambda i: (i, 0)),
        out_shape=jax.ShapeDtypeStruct((N, 64), jnp.float32),
    )(s2a, s2b, hp2, dinv_col)


def kernel(X, edge_index, W1, b1, W2, b2):
    e = edge_index.shape[1]
    src = edge_index[0]
    dst = edge_index[1]
    src3, dst3, n_chunk = _pad_edges(src, dst, e)

    b1r = b1.reshape(1, 128)
    b2r = b2.reshape(1, 64)
    z16 = jnp.zeros((ROWS_PER_TILE, 16), jnp.float32)
    z128 = jnp.zeros((ROWS_PER_TILE, 128), jnp.float32)
    z64 = jnp.zeros((ROWS_PER_TILE, 64), jnp.float32)

    src3h = src3.reshape(NW, 2 * n_chunk, CHW // 2)
    dst3h = dst3.reshape(NW, 2 * n_chunk, CHW // 2)

    degp = _make_deg_kernel(n_chunk)(dst3, z16)
    deg = degp[0, :N, 0] + degp[1, :N, 0] + 1.0
    dinv_col = lax.rsqrt(jnp.maximum(deg, 1.0))[:, None]

    hp1 = _mm1(X, W1, b1r, dinv_col)
    s1 = _make_scat_kernel(2 * n_chunk, CHW // 2, 128)(hp1, src3h, dst3h, z128)
    out1, hp2 = _mm2(s1[0], s1[1], hp1, dinv_col, W2, b2r)
    s2 = _make_scat_kernel(n_chunk, CHW, 64)(hp2, src3, dst3, z64)
    out2 = _fin(s2[0], s2[1], hp2, dinv_col)
    return (out1, out2)


# speedup vs baseline: 1.2031x; 1.2031x over previous
import functools

import jax
import jax.numpy as jnp
from jax import lax
from jax.experimental import pallas as pl
from jax.experimental.pallas import tpu as pltpu
from jax.experimental.pallas import tpu_sc as plsc

N = 10000
NP = 10240
NBLK = 5
BLK = N // NBLK
NW = 32
TPC = 16
ROWS_PER_TILE = NP // TPC
CHW = 128


def _pad_edges(src, dst, e, ch):
    epw = -(-e // NW)
    n_chunk = -(-epw // ch)
    if n_chunk % 2 == 0:
        n_chunk += 1
    ep = NW * n_chunk * ch
    pad = ep - e
    src3 = jnp.concatenate([src, jnp.zeros((pad,), jnp.int32)])
    dst3 = jnp.concatenate([dst, jnp.full((pad,), N, dtype=jnp.int32)])
    return src3.reshape(NW, n_chunk, ch), dst3.reshape(NW, n_chunk, ch), n_chunk


_SC_PARAMS = pltpu.CompilerParams(use_tc_tiling_on_sc=False)


def _make_deg_kernel(n_chunk):
    mesh = plsc.VectorSubcoreMesh(core_axis_name="c", subcore_axis_name="s")

    @functools.partial(
        pl.kernel,
        out_type=jax.ShapeDtypeStruct((2, NP, 16), jnp.float32),
        mesh=mesh,
        compiler_params=_SC_PARAMS,
        scratch_types=[
            pltpu.VMEM_SHARED((NP, 16), jnp.float32),
            pltpu.VMEM((n_chunk, CHW), jnp.int32),
            pltpu.VMEM((CHW, 16), jnp.float32),
        ],
    )
    def deg_kernel(dst3_hbm, zeros_hbm, out_hbm, acc, dstv, ones_v):
        c = lax.axis_index("c")
        s = lax.axis_index("s")
        wid = c * TPC + s
        pltpu.sync_copy(zeros_hbm, acc.at[pl.ds(s * ROWS_PER_TILE, ROWS_PER_TILE)])
        for r in range(CHW):
            ones_v[r, :] = jnp.ones((16,), jnp.float32)
        plsc.subcore_barrier()
        pltpu.sync_copy(dst3_hbm.at[wid], dstv)

        def chunk(j, carry):
            pltpu.sync_copy(ones_v, acc.at[dstv.at[j]], add=True)
            return carry

        lax.fori_loop(0, n_chunk, chunk, 0)
        plsc.subcore_barrier()
        sl = pl.ds(s * ROWS_PER_TILE, ROWS_PER_TILE)
        pltpu.sync_copy(acc.at[sl], out_hbm.at[c, sl])

    return deg_kernel


def _make_scat_kernel(n_chunk, ch, d):
    mesh = plsc.VectorSubcoreMesh(core_axis_name="c", subcore_axis_name="s")

    @functools.partial(
        pl.kernel,
        out_type=jax.ShapeDtypeStruct((2, NP, d), jnp.float32),
        mesh=mesh,
        compiler_params=_SC_PARAMS,
        scratch_types=[
            pltpu.VMEM_SHARED((NP, d), jnp.float32),
            pltpu.VMEM((n_chunk, ch), jnp.int32),
            pltpu.VMEM((n_chunk, ch), jnp.int32),
            pltpu.VMEM((ch, d), jnp.float32),
            pltpu.VMEM((ch, d), jnp.float32),
            pltpu.SemaphoreType.DMA,
            pltpu.SemaphoreType.DMA,
        ],
    )
    def scat_kernel(hp_hbm, src3_hbm, dst3_hbm, zeros_hbm, out_hbm,
                    acc, srcv, dstv, rows_a, rows_b, sem_a, sem_b):
        c = lax.axis_index("c")
        s = lax.axis_index("s")
        wid = c * TPC + s
        pltpu.sync_copy(zeros_hbm, acc.at[pl.ds(s * ROWS_PER_TILE, ROWS_PER_TILE)])
        plsc.subcore_barrier()
        pltpu.sync_copy(src3_hbm.at[wid], srcv)
        pltpu.sync_copy(dst3_hbm.at[wid], dstv)

        def g_start(j, buf, sem):
            pltpu.async_copy(hp_hbm.at[srcv.at[j]], buf, sem)

        def g_wait(j, buf, sem):
            pltpu.make_async_copy(hp_hbm.at[srcv.at[j]], buf, sem).wait()

        def scat(j, buf):
            pltpu.sync_copy(buf, acc.at[dstv.at[j]], add=True)

        g_start(0, rows_a, sem_a)

        def pair(k, carry):
            ja = 2 * k
            g_wait(ja, rows_a, sem_a)
            g_start(ja + 1, rows_b, sem_b)
            scat(ja, rows_a)
            g_wait(ja + 1, rows_b, sem_b)
            g_start(ja + 2, rows_a, sem_a)
            scat(ja + 1, rows_b)
            return carry

        if n_chunk % 2:
            lax.fori_loop(0, (n_chunk - 1) // 2, pair, 0)
            g_wait(n_chunk - 1, rows_a, sem_a)
            scat(n_chunk - 1, rows_a)
        else:
            lax.fori_loop(0, n_chunk // 2 - 1, pair, 0)
            g_wait(n_chunk - 2, rows_a, sem_a)
            g_start(n_chunk - 1, rows_b, sem_b)
            scat(n_chunk - 2, rows_a)
            g_wait(n_chunk - 1, rows_b, sem_b)
            scat(n_chunk - 1, rows_b)
        plsc.subcore_barrier()
        sl = pl.ds(s * ROWS_PER_TILE, ROWS_PER_TILE)
        pltpu.sync_copy(acc.at[sl], out_hbm.at[c, sl])

    return scat_kernel


def _mm1_body(x_ref, w_ref, b_ref, dinv_ref, o_ref):
    acc = jnp.dot(x_ref[...], w_ref[...], preferred_element_type=jnp.float32)
    o_ref[...] = dinv_ref[...] * (acc + b_ref[...])


def _mm1(x, w1, b1r, dinv_col):
    return pl.pallas_call(
        _mm1_body,
        grid=(NBLK,),
        in_specs=[
            pl.BlockSpec((BLK, 128), lambda i: (i, 0)),
            pl.BlockSpec((128, 128), lambda i: (0, 0)),
            pl.BlockSpec((1, 128), lambda i: (0, 0)),
            pl.BlockSpec((BLK, 1), lambda i: (i, 0)),
        ],
        out_specs=pl.BlockSpec((BLK, 128), lambda i: (i, 0)),
        out_shape=jax.ShapeDtypeStruct((N, 128), jnp.float32),
    )(x, w1, b1r, dinv_col)


def _mm2_body(sa_ref, sb_ref, hp_ref, dinv_ref, w_ref, b_ref, o1_ref, h2_ref):
    smooth = dinv_ref[...] * (sa_ref[...] + sb_ref[...] + hp_ref[...])
    o1 = jnp.maximum(smooth, 0.0)
    o1_ref[...] = o1
    acc = jnp.dot(o1, w_ref[...], preferred_element_type=jnp.float32)
    h2_ref[...] = dinv_ref[...] * (acc + b_ref[...])


def _mm2(s1a, s1b, hp1, dinv_col, w2, b2r):
    return pl.pallas_call(
        _mm2_body,
        grid=(NBLK,),
        in_specs=[
            pl.BlockSpec((BLK, 128), lambda i: (i, 0)),
            pl.BlockSpec((BLK, 128), lambda i: (i, 0)),
            pl.BlockSpec((BLK, 128), lambda i: (i, 0)),
            pl.BlockSpec((BLK, 1), lambda i: (i, 0)),
            pl.BlockSpec((128, 64), lambda i: (0, 0)),
            pl.BlockSpec((1, 64), lambda i: (0, 0)),
        ],
        out_specs=[
            pl.BlockSpec((BLK, 128), lambda i: (i, 0)),
            pl.BlockSpec((BLK, 64), lambda i: (i, 0)),
        ],
        out_shape=[
            jax.ShapeDtypeStruct((N, 128), jnp.float32),
            jax.ShapeDtypeStruct((N, 64), jnp.float32),
        ],
    )(s1a, s1b, hp1, dinv_col, w2, b2r)


def _fin_body(sa_ref, sb_ref, hp_ref, dinv_ref, o_ref):
    o_ref[...] = dinv_ref[...] * (sa_ref[...] + sb_ref[...] + hp_ref[...])


def _fin(s2a, s2b, hp2, dinv_col):
    return pl.pallas_call(
        _fin_body,
        grid=(NBLK,),
        in_specs=[
            pl.BlockSpec((BLK, 64), lambda i: (i, 0)),
            pl.BlockSpec((BLK, 64), lambda i: (i, 0)),
            pl.BlockSpec((BLK, 64), lambda i: (i, 0)),
            pl.BlockSpec((BLK, 1), lambda i: (i, 0)),
        ],
        out_specs=pl.BlockSpec((BLK, 64), lambda i: (i, 0)),
        out_shape=jax.ShapeDtypeStruct((N, 64), jnp.float32),
    )(s2a, s2b, hp2, dinv_col)


def kernel(X, edge_index, W1, b1, W2, b2):
    e = edge_index.shape[1]
    src = edge_index[0]
    dst = edge_index[1]
    src3w, dst3w, n_w = _pad_edges(src, dst, e, CHW)
    src3h, dst3h, n_h = _pad_edges(src, dst, e, CHW // 2)

    b1r = b1.reshape(1, 128)
    b2r = b2.reshape(1, 64)
    z16 = jnp.zeros((ROWS_PER_TILE, 16), jnp.float32)
    z128 = jnp.zeros((ROWS_PER_TILE, 128), jnp.float32)
    z64 = jnp.zeros((ROWS_PER_TILE, 64), jnp.float32)

    degp = _make_deg_kernel(n_w)(dst3w, z16)
    deg = degp[0, :N, 0] + degp[1, :N, 0] + 1.0
    dinv_col = lax.rsqrt(jnp.maximum(deg, 1.0))[:, None]

    hp1 = _mm1(X, W1, b1r, dinv_col)
    s1 = _make_scat_kernel(n_h, CHW // 2, 128)(hp1, src3h, dst3h, z128)
    out1, hp2 = _mm2(s1[0], s1[1], hp1, dinv_col, W2, b2r)
    s2 = _make_scat_kernel(n_w, CHW, 64)(hp2, src3w, dst3w, z64)
    out2 = _fin(s2[0], s2[1], hp2, dinv_col)
    return (out1, out2)


# speedup vs baseline: 1.2293x; 1.0218x over previous
import functools

import jax
import jax.numpy as jnp
from jax import lax
from jax.experimental import pallas as pl
from jax.experimental.pallas import tpu as pltpu
from jax.experimental.pallas import tpu_sc as plsc

N = 10000
NP = 10240
NBLK = 8
BLK = NP // NBLK
NW = 32
TPC = 16
ROWS_PER_TILE = NP // TPC
CH = 64


def _pad_edges(src, dst, e):
    epw = -(-e // NW)
    n_chunk = -(-epw // CH)
    if n_chunk % 2 == 0:
        n_chunk += 1
    ep = NW * n_chunk * CH
    pad = ep - e
    fill = jnp.full((pad,), N, dtype=jnp.int32)
    src3 = jnp.concatenate([src, fill]).reshape(NW, n_chunk, CH)
    dst3 = jnp.concatenate([dst, fill]).reshape(NW, n_chunk, CH)
    return src3, dst3, n_chunk


_SC_PARAMS = pltpu.CompilerParams(use_tc_tiling_on_sc=False)


def _make_deg_kernel(n_chunk):
    mesh = plsc.VectorSubcoreMesh(core_axis_name="c", subcore_axis_name="s")

    @functools.partial(
        pl.kernel,
        out_type=jax.ShapeDtypeStruct((2, NP, 16), jnp.float32),
        mesh=mesh,
        compiler_params=_SC_PARAMS,
        scratch_types=[
            pltpu.VMEM_SHARED((NP, 16), jnp.float32),
            pltpu.VMEM((n_chunk, CH), jnp.int32),
            pltpu.VMEM((CH, 16), jnp.float32),
        ],
    )
    def deg_kernel(dst3_hbm, zeros_hbm, out_hbm, acc, dstv, ones_v):
        c = lax.axis_index("c")
        s = lax.axis_index("s")
        wid = c * TPC + s
        pltpu.sync_copy(zeros_hbm, acc.at[pl.ds(s * ROWS_PER_TILE, ROWS_PER_TILE)])
        for r in range(CH):
            ones_v[r, :] = jnp.ones((16,), jnp.float32)
        plsc.subcore_barrier()
        pltpu.sync_copy(dst3_hbm.at[wid], dstv)

        def chunk(j, carry):
            pltpu.sync_copy(ones_v, acc.at[dstv.at[j]], add=True)
            return carry

        lax.fori_loop(0, n_chunk, chunk, 0)
        plsc.subcore_barrier()
        sl = pl.ds(s * ROWS_PER_TILE, ROWS_PER_TILE)
        pltpu.sync_copy(acc.at[sl], out_hbm.at[c, sl])

    return deg_kernel


def _make_scat_kernel(n_chunk, d):
    mesh = plsc.VectorSubcoreMesh(core_axis_name="c", subcore_axis_name="s")

    @functools.partial(
        pl.kernel,
        out_type=jax.ShapeDtypeStruct((2, NP, d), jnp.float32),
        mesh=mesh,
        compiler_params=_SC_PARAMS,
        scratch_types=[
            pltpu.VMEM_SHARED((NP, d), jnp.float32),
            pltpu.VMEM((n_chunk, CH), jnp.int32),
            pltpu.VMEM((n_chunk, CH), jnp.int32),
            pltpu.VMEM((CH, d), jnp.float32),
            pltpu.VMEM((CH, d), jnp.float32),
            pltpu.SemaphoreType.DMA,
            pltpu.SemaphoreType.DMA,
        ],
    )
    def scat_kernel(hp_hbm, src3_hbm, dst3_hbm, zeros_hbm, out_hbm,
                    acc, srcv, dstv, rows_a, rows_b, sem_a, sem_b):
        c = lax.axis_index("c")
        s = lax.axis_index("s")
        wid = c * TPC + s
        pltpu.sync_copy(zeros_hbm, acc.at[pl.ds(s * ROWS_PER_TILE, ROWS_PER_TILE)])
        plsc.subcore_barrier()
        pltpu.sync_copy(src3_hbm.at[wid], srcv)
        pltpu.sync_copy(dst3_hbm.at[wid], dstv)

        def g_start(j, buf, sem):
            pltpu.async_copy(hp_hbm.at[srcv.at[j]], buf, sem)

        def g_wait(j, buf, sem):
            pltpu.make_async_copy(hp_hbm.at[srcv.at[j]], buf, sem).wait()

        def scat(j, buf):
            pltpu.sync_copy(buf, acc.at[dstv.at[j]], add=True)

        n2 = (n_chunk - 1) // 2
        g_start(0, rows_a, sem_a)

        def pair(k, carry):
            ja = 2 * k
            g_wait(ja, rows_a, sem_a)
            g_start(ja + 1, rows_b, sem_b)
            scat(ja, rows_a)
            g_wait(ja + 1, rows_b, sem_b)
            g_start(ja + 2, rows_a, sem_a)
            scat(ja + 1, rows_b)
            return carry

        lax.fori_loop(0, n2, pair, 0)
        g_wait(n_chunk - 1, rows_a, sem_a)
        scat(n_chunk - 1, rows_a)
        plsc.subcore_barrier()
        sl = pl.ds(s * ROWS_PER_TILE, ROWS_PER_TILE)
        pltpu.sync_copy(acc.at[sl], out_hbm.at[c, sl])

    return scat_kernel


def _mm1_body(x_ref, w_ref, b_ref, dinv_ref, o_ref):
    acc = jnp.dot(x_ref[...], w_ref[...], preferred_element_type=jnp.float32)
    o_ref[...] = dinv_ref[...] * (acc + b_ref[...])


def _mm1(xp, w1, b1r, dinv_col):
    return pl.pallas_call(
        _mm1_body,
        grid=(NBLK,),
        in_specs=[
            pl.BlockSpec((BLK, 128), lambda i: (i, 0)),
            pl.BlockSpec((128, 128), lambda i: (0, 0)),
            pl.BlockSpec((1, 128), lambda i: (0, 0)),
            pl.BlockSpec((BLK, 1), lambda i: (i, 0)),
        ],
        out_specs=pl.BlockSpec((BLK, 128), lambda i: (i, 0)),
        out_shape=jax.ShapeDtypeStruct((NP, 128), jnp.float32),
    )(xp, w1, b1r, dinv_col)


def _mm2_body(sa_ref, sb_ref, hp_ref, dinv_ref, w_ref, b_ref, o1_ref, h2_ref):
    smooth = dinv_ref[...] * (sa_ref[...] + sb_ref[...] + hp_ref[...])
    o1 = jnp.maximum(smooth, 0.0)
    o1_ref[...] = o1
    acc = jnp.dot(o1, w_ref[...], preferred_element_type=jnp.float32)
    h2_ref[...] = dinv_ref[...] * (acc + b_ref[...])


def _mm2(s1a, s1b, hp1, dinv_col, w2, b2r):
    return pl.pallas_call(
        _mm2_body,
        grid=(NBLK,),
        in_specs=[
            pl.BlockSpec((BLK, 128), lambda i: (i, 0)),
            pl.BlockSpec((BLK, 128), lambda i: (i, 0)),
            pl.BlockSpec((BLK, 128), lambda i: (i, 0)),
            pl.BlockSpec((BLK, 1), lambda i: (i, 0)),
            pl.BlockSpec((128, 64), lambda i: (0, 0)),
            pl.BlockSpec((1, 64), lambda i: (0, 0)),
        ],
        out_specs=[
            pl.BlockSpec((BLK, 128), lambda i: (i, 0)),
            pl.BlockSpec((BLK, 64), lambda i: (i, 0)),
        ],
        out_shape=[
            jax.ShapeDtypeStruct((NP, 128), jnp.float32),
            jax.ShapeDtypeStruct((NP, 64), jnp.float32),
        ],
    )(s1a, s1b, hp1, dinv_col, w2, b2r)


def _fin_body(sa_ref, sb_ref, hp_ref, dinv_ref, o_ref):
    o_ref[...] = dinv_ref[...] * (sa_ref[...] + sb_ref[...] + hp_ref[...])


def _fin(s2a, s2b, hp2, dinv_col):
    return pl.pallas_call(
        _fin_body,
        grid=(NBLK,),
        in_specs=[
            pl.BlockSpec((BLK, 64), lambda i: (i, 0)),
            pl.BlockSpec((BLK, 64), lambda i: (i, 0)),
            pl.BlockSpec((BLK, 64), lambda i: (i, 0)),
            pl.BlockSpec((BLK, 1), lambda i: (i, 0)),
        ],
        out_specs=pl.BlockSpec((BLK, 64), lambda i: (i, 0)),
        out_shape=jax.ShapeDtypeStruct((NP, 64), jnp.float32),
    )(s2a, s2b, hp2, dinv_col)


def kernel(X, edge_index, W1, b1, W2, b2):
    e = edge_index.shape[1]
    src = edge_index[0]
    dst = edge_index[1]
    src3, dst3, n_chunk = _pad_edges(src, dst, e)

    xp = jnp.zeros((NP, 128), jnp.float32).at[:N].set(X)
    b1r = b1.reshape(1, 128)
    b2r = b2.reshape(1, 64)
    z16 = jnp.zeros((ROWS_PER_TILE, 16), jnp.float32)
    z128 = jnp.zeros((ROWS_PER_TILE, 128), jnp.float32)
    z64 = jnp.zeros((ROWS_PER_TILE, 64), jnp.float32)

    degp = _make_deg_kernel(n_chunk)(dst3, z16)
    deg = degp[0, :, 0] + degp[1, :, 0] + 1.0
    dinv = lax.rsqrt(jnp.maximum(deg, 1.0))
    dinv = jnp.where(jnp.arange(NP) < N, dinv, 0.0)
    dinv_col = dinv[:, None]

    hp1 = _mm1(xp, W1, b1r, dinv_col)
    s1 = _make_scat_kernel(n_chunk, 128)(hp1, src3, dst3, z128)
    out1p, hp2 = _mm2(s1[0], s1[1], hp1, dinv_col, W2, b2r)
    s2 = _make_scat_kernel(n_chunk, 64)(hp2, src3, dst3, z64)
    out2p = _fin(s2[0], s2[1], hp2, dinv_col)
    return (out1p[:N], out2p[:N])
